# baseline (speedup 1.0000x reference)
import jax
import jax.numpy as jnp
from jax import lax
from jax.experimental import pallas as pl
from jax.experimental.pallas import tpu as pltpu
from jax.experimental.pallas import tpu_sc as plsc

NC, NS = 2, 16
NW = NC * NS
B = 16384
D = 128
BPW = B // NW
C = 5
R = 64
NCHUNK = BPW // R
NV = 16
WBN = C * 2 * D + NV


def _fused_body(user_t, game_t, user_idx, game_idx, wb_h, out_h,
                idxu, idxg, ub0, ub1, gb0, gb1, wv, part, outv,
                su0, su1, sg0, sg1):
    wid = lax.axis_index("s") * NC + lax.axis_index("c")
    base = wid * BPW
    pltpu.sync_copy(user_idx.at[pl.ds(base, BPW)], idxu)
    pltpu.sync_copy(game_idx.at[pl.ds(base, BPW)], idxg)
    pltpu.sync_copy(wb_h, wv)

    ubufs, gbufs = (ub0, ub1), (gb0, gb1)
    usems, gsems = (su0, su1), (sg0, sg1)

    def start(k):
        s = k % 2
        cu = pltpu.async_copy(user_t.at[idxu.at[pl.ds(k * R, R)]],
                              ubufs[s], usems[s])
        cg = pltpu.async_copy(game_t.at[idxg.at[pl.ds(k * R, R)]],
                              gbufs[s], gsems[s])
        return cu, cg

    pend = start(0)
    for k in range(NCHUNK):
        nxt = start(k + 1) if k + 1 < NCHUNK else None
        pend[0].wait()
        pend[1].wait()
        for half, buf in ((0, ubufs[k % 2]), (1, gbufs[k % 2])):
            wregs = [[wv[pl.ds(c * 2 * D + half * D + NV * j, NV)]
                      for j in range(8)] for c in range(C)]

            @plsc.parallel_loop(0, R, unroll=2)
            def mac_body(b, wregs=wregs, buf=buf, half=half, k=k):
                rows = [buf[b, pl.ds(NV * j, NV)] for j in range(8)]
                for c in range(C):
                    acc = [rows[j] * wregs[c][j] for j in range(8)]
                    while len(acc) > 1:
                        acc = [acc[i] + acc[i + 1]
                               for i in range(0, len(acc), 2)]
                    t = acc[0]
                    if half:
                        t = t + part[pl.ds((k * R + b) * C * NV + c * NV, NV)]
                    part[pl.ds((k * R + b) * C * NV + c * NV, NV)] = t
        pend = nxt

    lanes = lax.iota(jnp.int32, NV)
    bias_vec = wv[pl.ds(C * 2 * D, NV)]
    out_mask = lanes < C

    @plsc.parallel_loop(0, BPW, unroll=4)
    def red_body(b):
        v = bias_vec
        for c in range(C):
            tot = plsc.cumsum(part[pl.ds(b * C * NV + c * NV, NV)])[NV - 1]
            v = jnp.where(lanes == c, bias_vec + tot, v)
        plsc.store_scatter(outv, [b * C + lanes], v, mask=out_mask)

    pltpu.sync_copy(outv, out_h.at[pl.ds(base * C, BPW * C)])


_sc_fused = pl.kernel(
    _fused_body,
    out_type=jax.ShapeDtypeStruct((B * C,), jnp.float32),
    mesh=plsc.VectorSubcoreMesh(core_axis_name="c", subcore_axis_name="s"),
    compiler_params=pltpu.CompilerParams(needs_layout_passes=False),
    scratch_types=[
        pltpu.VMEM((BPW,), jnp.int32),
        pltpu.VMEM((BPW,), jnp.int32),
        pltpu.VMEM((R, D), jnp.float32),
        pltpu.VMEM((R, D), jnp.float32),
        pltpu.VMEM((R, D), jnp.float32),
        pltpu.VMEM((R, D), jnp.float32),
        pltpu.VMEM((WBN,), jnp.float32),
        pltpu.VMEM((BPW * C * NV,), jnp.float32),
        pltpu.VMEM((BPW * C,), jnp.float32),
        pltpu.SemaphoreType.DMA,
        pltpu.SemaphoreType.DMA,
        pltpu.SemaphoreType.DMA,
        pltpu.SemaphoreType.DMA,
    ],
)


def kernel(user, game, user_table, game_table, fc_w, fc_b):
    wb = jnp.concatenate([fc_w.reshape(-1), fc_b,
                          jnp.zeros((NV - C,), jnp.float32)])
    return _sc_fused(user_table, game_table, user, game, wb).reshape(B, C)


# speedup vs baseline: 1.6903x; 1.6903x over previous
import jax
import jax.numpy as jnp
from jax import lax
from jax.experimental import pallas as pl
from jax.experimental.pallas import tpu as pltpu
from jax.experimental.pallas import tpu_sc as plsc

NC, NS = 2, 16
NW = NC * NS
B = 16384
D = 128
BPW = B // NW
C = 5


def _gather_body(user_t, game_t, user_idx, game_idx, uout, gout,
                 idx_v, rows_v, sem):
    wid = lax.axis_index("s") * NC + lax.axis_index("c")
    base = wid * BPW
    pltpu.sync_copy(user_idx.at[pl.ds(base, BPW)], idx_v)
    pltpu.async_copy(user_t.at[idx_v], rows_v, sem).wait()
    pltpu.sync_copy(rows_v, uout.at[pl.ds(base, BPW)])
    pltpu.sync_copy(game_idx.at[pl.ds(base, BPW)], idx_v)
    pltpu.async_copy(game_t.at[idx_v], rows_v, sem).wait()
    pltpu.sync_copy(rows_v, gout.at[pl.ds(base, BPW)])


_sc_gather = pl.kernel(
    _gather_body,
    out_type=(jax.ShapeDtypeStruct((B, D), jnp.float32),
              jax.ShapeDtypeStruct((B, D), jnp.float32)),
    mesh=plsc.VectorSubcoreMesh(core_axis_name="c", subcore_axis_name="s"),
    scratch_types=[
        pltpu.VMEM((BPW,), jnp.int32),
        pltpu.VMEM((BPW, D), jnp.float32),
        pltpu.SemaphoreType.DMA,
    ],
)

_DN = (((1,), (1,)), ((), ()))


def _matmul_body(u_ref, g_ref, w_ref, b_ref, o_ref):
    w = w_ref[...]
    acc = lax.dot_general(u_ref[...], w[:, :D], _DN,
                          preferred_element_type=jnp.float32)
    acc += lax.dot_general(g_ref[...], w[:, D:], _DN,
                           preferred_element_type=jnp.float32)
    o_ref[...] = acc + b_ref[...]


def _tc_matmul(uemb, gemb, fc_w, bias):
    bm = 4096
    grid = (B // bm,)
    return pl.pallas_call(
        _matmul_body,
        grid=grid,
        in_specs=[
            pl.BlockSpec((bm, D), lambda i: (i, 0)),
            pl.BlockSpec((bm, D), lambda i: (i, 0)),
            pl.BlockSpec((C, 2 * D), lambda i: (0, 0)),
            pl.BlockSpec((1, C), lambda i: (0, 0)),
        ],
        out_specs=pl.BlockSpec((bm, C), lambda i: (i, 0)),
        out_shape=jax.ShapeDtypeStruct((B, C), jnp.float32),
    )(uemb, gemb, fc_w, bias)


def kernel(user, game, user_table, game_table, fc_w, fc_b):
    uemb, gemb = _sc_gather(user_table, game_table, user, game)
    return _tc_matmul(uemb, gemb, fc_w, fc_b.reshape(1, C))
